# baseline (speedup 1.0000x reference)
import jax
import jax.numpy as jnp
from jax.experimental import pallas as pl
from jax.experimental.pallas import tpu as pltpu

_POS_W = 2.0
_BR = 4
_NBUF = 6


def _make_body(n_blocks, cols):
    def body(o_hbm, t_hbm, acc_ref, obuf, tbuf, sems):
        def copy_pair(b):
            s = b % _NBUF
            oc = pltpu.make_async_copy(
                o_hbm.at[pl.ds(b * _BR, _BR), :], obuf.at[s], sems.at[0, s]
            )
            tc = pltpu.make_async_copy(
                t_hbm.at[pl.ds(b * _BR, _BR), :], tbuf.at[s], sems.at[1, s]
            )
            return oc, tc

        for b in range(_NBUF):
            oc, tc = copy_pair(b)
            oc.start()
            tc.start()

        total = jnp.float32(0.0)
        for b in range(n_blocks):
            s = b % _NBUF
            oc, tc = copy_pair(b)
            oc.wait()
            tc.wait()
            o = obuf[s]
            t = tbuf[s]
            w = jnp.where(t > 0, -_POS_W * o, jnp.where(t < 0, o, 0.0))
            total = total + jnp.sum(w)
            if b + _NBUF < n_blocks:
                oc, tc = copy_pair(b + _NBUF)
                oc.start()
                tc.start()
        acc_ref[0, 0] = total

    return body


def kernel(output, target):
    rows, cols = output.shape
    n_blocks = rows // _BR
    res = pl.pallas_call(
        _make_body(n_blocks, cols),
        in_specs=[
            pl.BlockSpec(memory_space=pl.ANY),
            pl.BlockSpec(memory_space=pl.ANY),
        ],
        out_specs=pl.BlockSpec(memory_space=pltpu.SMEM),
        out_shape=jax.ShapeDtypeStruct((1, 1), jnp.float32),
        scratch_shapes=[
            pltpu.VMEM((_NBUF, _BR, 32768), jnp.float32),
            pltpu.VMEM((_NBUF, _BR, 32768), jnp.float32),
            pltpu.SemaphoreType.DMA((2, _NBUF)),
        ],
    )(output, target)
    return res[0, 0]


# speedup vs baseline: 1.2324x; 1.2324x over previous
import jax
import jax.numpy as jnp
from jax.experimental import pallas as pl
from jax.experimental.pallas import tpu as pltpu

_POS_W = 2.0
_BR = 16
_NBUF = 4


def _make_body(n_blocks, cols):
    def body(o_hbm, t_hbm, acc_ref, obuf, tbuf, sems):
        def copy_pair(b):
            s = b % _NBUF
            oc = pltpu.make_async_copy(
                o_hbm.at[pl.ds(b * _BR, _BR), :], obuf.at[s], sems.at[0, s]
            )
            tc = pltpu.make_async_copy(
                t_hbm.at[pl.ds(b * _BR, _BR), :], tbuf.at[s], sems.at[1, s]
            )
            return oc, tc

        for b in range(_NBUF):
            oc, tc = copy_pair(b)
            oc.start()
            tc.start()

        total = jnp.float32(0.0)
        for b in range(n_blocks):
            s = b % _NBUF
            oc, tc = copy_pair(b)
            oc.wait()
            tc.wait()
            o = obuf[s]
            t = tbuf[s]
            w = jnp.where(t > 0, -_POS_W * o, jnp.where(t < 0, o, 0.0))
            total = total + jnp.sum(w)
            if b + _NBUF < n_blocks:
                oc, tc = copy_pair(b + _NBUF)
                oc.start()
                tc.start()
        acc_ref[0, 0] = total

    return body


def kernel(output, target):
    rows, cols = output.shape
    n_blocks = rows // _BR
    res = pl.pallas_call(
        _make_body(n_blocks, cols),
        in_specs=[
            pl.BlockSpec(memory_space=pl.ANY),
            pl.BlockSpec(memory_space=pl.ANY),
        ],
        out_specs=pl.BlockSpec(memory_space=pltpu.SMEM),
        out_shape=jax.ShapeDtypeStruct((1, 1), jnp.float32),
        scratch_shapes=[
            pltpu.VMEM((_NBUF, _BR, 32768), jnp.float32),
            pltpu.VMEM((_NBUF, _BR, 32768), jnp.float32),
            pltpu.SemaphoreType.DMA((2, _NBUF)),
        ],
    )(output, target)
    return res[0, 0]


# speedup vs baseline: 1.2582x; 1.0210x over previous
import jax
import jax.numpy as jnp
from jax.experimental import pallas as pl
from jax.experimental.pallas import tpu as pltpu

_POS_W = 2.0
_BR = 8
_NBUF = 4
_NSPLIT = 2


def _make_body(n_blocks, cols):
    half = cols // _NSPLIT

    def body(o_hbm, t_hbm, acc_ref, obuf, tbuf, sems):
        def copies(b):
            s = b % _NBUF
            cps = []
            for h in range(_NSPLIT):
                csl = pl.ds(h * half, half)
                cps.append(pltpu.make_async_copy(
                    o_hbm.at[pl.ds(b * _BR, _BR), csl],
                    obuf.at[s, :, csl],
                    sems.at[0, s],
                ))
                cps.append(pltpu.make_async_copy(
                    t_hbm.at[pl.ds(b * _BR, _BR), csl],
                    tbuf.at[s, :, csl],
                    sems.at[1, s],
                ))
            return cps

        for b in range(_NBUF):
            for cp in copies(b):
                cp.start()

        total = jnp.float32(0.0)
        for b in range(n_blocks):
            s = b % _NBUF
            for cp in copies(b):
                cp.wait()
            o = obuf[s]
            t = tbuf[s]
            w = jnp.where(t > 0, -_POS_W * o, jnp.where(t < 0, o, 0.0))
            total = total + jnp.sum(w)
            if b + _NBUF < n_blocks:
                for cp in copies(b + _NBUF):
                    cp.start()
        acc_ref[0, 0] = total

    return body


def kernel(output, target):
    rows, cols = output.shape
    n_blocks = rows // _BR
    res = pl.pallas_call(
        _make_body(n_blocks, cols),
        in_specs=[
            pl.BlockSpec(memory_space=pl.ANY),
            pl.BlockSpec(memory_space=pl.ANY),
        ],
        out_specs=pl.BlockSpec(memory_space=pltpu.SMEM),
        out_shape=jax.ShapeDtypeStruct((1, 1), jnp.float32),
        scratch_shapes=[
            pltpu.VMEM((_NBUF, _BR, 32768), jnp.float32),
            pltpu.VMEM((_NBUF, _BR, 32768), jnp.float32),
            pltpu.SemaphoreType.DMA((2, _NBUF)),
        ],
    )(output, target)
    return res[0, 0]
